# baseline (speedup 1.0000x reference)
import jax
import jax.numpy as jnp
from jax.experimental import pallas as pl

N_TOKENS = 32768
D_MODEL = 768
NUM_EXPERTS = 64
BLK = 4096


def _router_body(x_ref, w_ref, b_ref, wtop_ref, idx_ref, soft_ref):
    x = x_ref[...]
    w = w_ref[...]
    logits = jax.lax.dot_general(
        x, w, (((1,), (0,)), ((), ())), preferred_element_type=jnp.float32)
    logits = logits + b_ref[...]
    m = jnp.max(logits, axis=-1, keepdims=True)
    e = jnp.exp(logits - m)
    s = jnp.sum(e, axis=-1, keepdims=True)
    soft = e / s
    soft_ref[...] = soft

    lane = jax.lax.broadcasted_iota(jnp.int32, soft.shape, 1)
    m1 = jnp.max(soft, axis=-1, keepdims=True)
    i1 = jnp.min(jnp.where(soft == m1, lane, NUM_EXPERTS), axis=-1,
                 keepdims=True)
    masked = jnp.where(lane == i1, -jnp.inf, soft)
    m2 = jnp.max(masked, axis=-1, keepdims=True)
    i2 = jnp.min(jnp.where(masked == m2, lane, NUM_EXPERTS), axis=-1,
                 keepdims=True)
    tot = m1 + m2
    wtop_ref[...] = jnp.concatenate([m1 / tot, m2 / tot], axis=-1)
    idx_ref[...] = jnp.concatenate([i1, i2], axis=-1)


@jax.jit
def kernel(x, W, b):
    n = x.shape[0]
    grid = (n // BLK,)
    wtop, idx, soft = pl.pallas_call(
        _router_body,
        grid=grid,
        in_specs=[
            pl.BlockSpec((BLK, D_MODEL), lambda i: (i, 0)),
            pl.BlockSpec((D_MODEL, NUM_EXPERTS), lambda i: (0, 0)),
            pl.BlockSpec((1, NUM_EXPERTS), lambda i: (0, 0)),
        ],
        out_specs=[
            pl.BlockSpec((BLK, 2), lambda i: (i, 0)),
            pl.BlockSpec((BLK, 2), lambda i: (i, 0)),
            pl.BlockSpec((BLK, NUM_EXPERTS), lambda i: (i, 0)),
        ],
        out_shape=[
            jax.ShapeDtypeStruct((n, 2), jnp.float32),
            jax.ShapeDtypeStruct((n, 2), jnp.int32),
            jax.ShapeDtypeStruct((n, NUM_EXPERTS), jnp.float32),
        ],
    )(x, W, b.reshape(1, NUM_EXPERTS))
    return (wtop, idx, soft)
